# baseline (speedup 1.0000x reference)
import functools

import jax
import jax.numpy as jnp
from jax import lax
from jax.experimental import pallas as pl
from jax.experimental.pallas import tpu as pltpu
from jax.experimental.pallas import tpu_sc as plsc

B = 16384
D = 16
NC = 2
NS = 16
NW = NC * NS
BPW = B // NW
G = 16
NG = BPW // G
RB = G * 8
EPS2 = 1e-16


def _rsqrt(x):
    i = lax.bitcast_convert_type(x, jnp.int32)
    i = jnp.int32(0x5F3759DF) - lax.shift_right_logical(i, 1)
    y = lax.bitcast_convert_type(i, jnp.float32)
    for _ in range(3):
        y = y * (1.5 - 0.5 * x * y * y)
    return y


def _body(uidx_hbm, midx_hbm, utab_hbm, mtab_hbm, out_hbm,
          uidx_v, midx_v, ublk_v, mblk_v, uring_v, mring_v, out_v,
          usem0, usem1, msem0, msem1, isem):
    wid = lax.axis_index("s") * NC + lax.axis_index("c")
    base = wid * BPW
    usems = (usem0, usem1)
    msems = (msem0, msem1)

    icopies = []
    for j in range(BPW // 128):
        icopies.append(pltpu.async_copy(
            uidx_hbm.at[pl.ds(base + j * 128, 128)],
            uidx_v.at[pl.ds(j * 128, 128)], isem))
        icopies.append(pltpu.async_copy(
            midx_hbm.at[pl.ds(base + j * 128, 128)],
            midx_v.at[pl.ds(j * 128, 128)], isem))
    for c in icopies:
        c.wait()

    for o in range(BPW // 16):
        s = pl.ds(o * 16, 16)
        ui = uidx_v[s]
        mi = midx_v[s]
        ublk_v[s] = ui & ~jnp.int32(7)
        mblk_v[s] = mi & ~jnp.int32(7)
        uidx_v[s] = ui & 7
        midx_v[s] = mi & 7

    lanes = lax.iota(jnp.int32, 16)

    def fire(g, bank):
        ubv = ublk_v[pl.ds(g * G, 16)]
        mbv = mblk_v[pl.ds(g * G, 16)]

        def one(j, carry):
            msk = lanes == j
            ub = lax.reduce_max(jnp.where(msk, ubv, 0), axes=(0,))
            mb = lax.reduce_max(jnp.where(msk, mbv, 0), axes=(0,))
            ub = pl.multiple_of(ub, 8)
            mb = pl.multiple_of(mb, 8)
            dst = pl.multiple_of(bank * RB + j * 8, 8)
            pltpu.async_copy(
                utab_hbm.at[pl.ds(ub, 8)], uring_v.at[pl.ds(dst, 8)],
                usems[bank])
            pltpu.async_copy(
                mtab_hbm.at[pl.ds(mb, 8)], mring_v.at[pl.ds(dst, 8)],
                msems[bank])
            return carry

        lax.fori_loop(0, G, one, jnp.int32(0))

    def drain(bank):
        boff = bank * RB
        pltpu.make_async_copy(
            utab_hbm.at[pl.ds(0, RB)], uring_v.at[pl.ds(boff, RB)],
            usems[bank]).wait()
        pltpu.make_async_copy(
            mtab_hbm.at[pl.ds(0, RB)], mring_v.at[pl.ds(boff, RB)],
            msems[bank]).wait()

    def compute(g, bank):
        s = pl.ds(g * G, 16)
        usub = uidx_v[s]
        msub = midx_v[s]
        urow = bank * RB + lanes * 8 + usub
        mrow = bank * RB + lanes * 8 + msub
        acc_um = jnp.zeros((16,), jnp.float32)
        acc_uu = jnp.zeros((16,), jnp.float32)
        acc_mm = jnp.zeros((16,), jnp.float32)
        for d in range(D):
            dv = jnp.full((16,), d, jnp.int32)
            u = plsc.load_gather(uring_v, [urow, dv])
            m = plsc.load_gather(mring_v, [mrow, dv])
            acc_um = acc_um + u * m
            acc_uu = acc_uu + u * u
            acc_mm = acc_mm + m * m
        denom2 = jnp.maximum(acc_uu, EPS2) * jnp.maximum(acc_mm, EPS2)
        sim = acc_um * _rsqrt(denom2) * 2.5 + 2.75
        out_v[pl.ds(g * G, 16)] = sim

    fire(jnp.int32(0), 0)
    fire(jnp.int32(1), 1)

    def step(i, carry):
        g0 = 2 * i
        for bank in range(2):
            g = g0 + bank
            drain(bank)
            compute(g, bank)

            @pl.when(g + 2 < NG)
            def _(g=g, bank=bank):
                fire(g + 2, bank)

        return carry

    lax.fori_loop(0, NG // 2, step, jnp.int32(0))

    pltpu.sync_copy(out_v, out_hbm.at[pl.ds(base, BPW)])


_mesh = plsc.VectorSubcoreMesh(core_axis_name="c", subcore_axis_name="s")

_sc_call = functools.partial(
    pl.kernel,
    mesh=_mesh,
    compiler_params=pltpu.CompilerParams(needs_layout_passes=False),
    out_type=jax.ShapeDtypeStruct((B,), jnp.float32),
    scratch_types=[
        pltpu.VMEM((BPW,), jnp.int32),
        pltpu.VMEM((BPW,), jnp.int32),
        pltpu.VMEM((BPW,), jnp.int32),
        pltpu.VMEM((BPW,), jnp.int32),
        pltpu.VMEM((2 * RB, D), jnp.float32),
        pltpu.VMEM((2 * RB, D), jnp.float32),
        pltpu.VMEM((BPW,), jnp.float32),
        pltpu.SemaphoreType.DMA,
        pltpu.SemaphoreType.DMA,
        pltpu.SemaphoreType.DMA,
        pltpu.SemaphoreType.DMA,
        pltpu.SemaphoreType.DMA,
    ],
)(_body)


def kernel(user_idx, movie_idx, user_table, movie_table):
    return _sc_call(user_idx.astype(jnp.int32), movie_idx.astype(jnp.int32),
                    user_table, movie_table)
